# Initial kernel scaffold; baseline (speedup 1.0000x reference)
import jax
import jax.numpy as jnp
from jax.experimental import pallas as pl


def kernel(feature, edge_index, W1, b1, W2, b2, W3, b3):
    raise NotImplementedError("write your pallas kernel here")


# speedup vs baseline: 3.7660x; 3.7660x over previous
import functools

import jax
import jax.numpy as jnp
from jax import lax
from jax.experimental import pallas as pl
from jax.experimental.pallas import tpu as pltpu
from jax.experimental.pallas import tpu_sc as plsc

N = 10000
E = 160000
D_IN = 256
D_HID = 128

NC = 2
NS = 16
NW = NC * NS
N_PAD = 10240
CE = 128
E_PAD = 163840
CHUNKS_W = E_PAD // (NW * CE)
ROWS_T = N_PAD // NS
ZR = 64

BR = 512
GRID = N_PAD // BR


def _mesh():
    return plsc.VectorSubcoreMesh(core_axis_name="c", subcore_axis_name="s")


@functools.partial(
    pl.kernel,
    out_type=(jax.ShapeDtypeStruct((NC, N_PAD, 16), jnp.float32),
              jax.ShapeDtypeStruct((NC, N_PAD, 16), jnp.float32)),
    mesh=_mesh(),
    scratch_types=[
        pltpu.VMEM((CHUNKS_W, CE), jnp.int32),
        pltpu.VMEM((CHUNKS_W, CE), jnp.int32),
        pltpu.VMEM((CE, 16), jnp.float32),
        pltpu.VMEM((ZR, 16), jnp.float32),
        pltpu.VMEM_SHARED((N_PAD, 16), jnp.float32),
        pltpu.VMEM_SHARED((N_PAD, 16), jnp.float32),
    ],
    name="sc_degrees",
    compiler_params=pltpu.CompilerParams(use_tc_tiling_on_sc=False),
)
def _deg_kernel(src_hbm, dst_hbm, ones_hbm, zeros_hbm, dout_hbm, din_hbm,
                srcv, dstv, ones_v, zeros_v, acc_o, acc_i):
    cid = lax.axis_index("c")
    sid = lax.axis_index("s")
    wid = cid * NS + sid
    pltpu.sync_copy(ones_hbm, ones_v)
    pltpu.sync_copy(zeros_hbm, zeros_v)

    def zbody(i, _):
        r0 = sid * ROWS_T + i * ZR
        pltpu.sync_copy(zeros_v, acc_o.at[pl.ds(r0, ZR)])
        pltpu.sync_copy(zeros_v, acc_i.at[pl.ds(r0, ZR)])
        return 0
    lax.fori_loop(0, ROWS_T // ZR, zbody, 0)

    pltpu.sync_copy(src_hbm.at[pl.ds(wid * CHUNKS_W, CHUNKS_W)], srcv)
    pltpu.sync_copy(dst_hbm.at[pl.ds(wid * CHUNKS_W, CHUNKS_W)], dstv)
    plsc.subcore_barrier()

    def body(k, _):
        pltpu.sync_copy(ones_v, acc_o.at[srcv.at[k]], add=True)
        pltpu.sync_copy(ones_v, acc_i.at[dstv.at[k]], add=True)
        return 0
    lax.fori_loop(0, CHUNKS_W, body, 0)
    plsc.subcore_barrier()

    r0 = sid * ROWS_T
    pltpu.sync_copy(acc_o.at[pl.ds(r0, ROWS_T)], dout_hbm.at[cid, pl.ds(r0, ROWS_T)])
    pltpu.sync_copy(acc_i.at[pl.ds(r0, ROWS_T)], din_hbm.at[cid, pl.ds(r0, ROWS_T)])


def _make_prop(D):
    @functools.partial(
        pl.kernel,
        out_type=jax.ShapeDtypeStruct((NC, N_PAD, D), jnp.float32),
        mesh=_mesh(),
        scratch_types=[
            pltpu.VMEM((CHUNKS_W, CE), jnp.int32),
            pltpu.VMEM((CHUNKS_W, CE), jnp.int32),
            pltpu.VMEM((CE, D), jnp.float32),
            pltpu.VMEM((ZR, D), jnp.float32),
            pltpu.SemaphoreType.DMA,
            pltpu.VMEM_SHARED((N_PAD, D), jnp.float32),
        ],
        name=f"sc_prop_{D}",
        compiler_params=pltpu.CompilerParams(use_tc_tiling_on_sc=(D == D_HID)),
    )
    def _prop(tbl_hbm, src_hbm, dst_hbm, zeros_hbm, out_hbm,
              srcv, dstv, rows_v, zeros_v, sem, acc):
        cid = lax.axis_index("c")
        sid = lax.axis_index("s")
        wid = cid * NS + sid
        pltpu.sync_copy(zeros_hbm, zeros_v)

        def zbody(i, _):
            pltpu.sync_copy(zeros_v, acc.at[pl.ds(sid * ROWS_T + i * ZR, ZR)])
            return 0
        lax.fori_loop(0, ROWS_T // ZR, zbody, 0)

        pltpu.sync_copy(src_hbm.at[pl.ds(wid * CHUNKS_W, CHUNKS_W)], srcv)
        pltpu.sync_copy(dst_hbm.at[pl.ds(wid * CHUNKS_W, CHUNKS_W)], dstv)
        plsc.subcore_barrier()

        def body(k, _):
            pltpu.async_copy(tbl_hbm.at[srcv.at[k]], rows_v, sem).wait()
            pltpu.sync_copy(rows_v, acc.at[dstv.at[k]], add=True)
            return 0
        lax.fori_loop(0, CHUNKS_W, body, 0)
        plsc.subcore_barrier()

        r0 = sid * ROWS_T
        pltpu.sync_copy(acc.at[pl.ds(r0, ROWS_T)], out_hbm.at[cid, pl.ds(r0, ROWS_T)])
    return _prop


_prop128 = _make_prop(D_HID)
_prop16 = _make_prop(16)


def _norms_body(do0, do1, di0, di1, ns_ref, nd_ref):
    ns_ref[...] = lax.rsqrt(jnp.maximum(do0[...] + do1[...], 1.0))
    nd_ref[...] = lax.rsqrt(jnp.maximum(di0[...] + di1[...], 1.0))


def _norms(do_p, di_p):
    return pl.pallas_call(
        _norms_body,
        out_shape=(jax.ShapeDtypeStruct((N_PAD, 16), jnp.float32),
                   jax.ShapeDtypeStruct((N_PAD, 16), jnp.float32)),
    )(do_p[0], do_p[1], di_p[0], di_p[1])


def _stage1_body(x_ref, w_ref, ns_ref, o_ref):
    o_ref[...] = jnp.dot(x_ref[...], w_ref[...],
                         preferred_element_type=jnp.float32) * ns_ref[:, 0:1]


def _stage1(x, w, ns):
    return pl.pallas_call(
        _stage1_body,
        grid=(GRID,),
        in_specs=[
            pl.BlockSpec((BR, D_IN), lambda i: (i, 0)),
            pl.BlockSpec((D_IN, D_HID), lambda i: (0, 0)),
            pl.BlockSpec((BR, 16), lambda i: (i, 0)),
        ],
        out_specs=pl.BlockSpec((BR, D_HID), lambda i: (i, 0)),
        out_shape=jax.ShapeDtypeStruct((N_PAD, D_HID), jnp.float32),
    )(x, w, ns)


def _stage_mid_body(p0_ref, p1_ref, nd_ref, ns_ref, b_ref, w_ref, o_ref):
    h = (p0_ref[...] + p1_ref[...]) * nd_ref[:, 0:1] + b_ref[...]
    o_ref[...] = jnp.dot(h, w_ref[...],
                         preferred_element_type=jnp.float32) * ns_ref[:, 0:1]


def _stage_mid(p0, p1, nd, ns, b, w, d_out):
    return pl.pallas_call(
        _stage_mid_body,
        grid=(GRID,),
        in_specs=[
            pl.BlockSpec((BR, D_HID), lambda i: (i, 0)),
            pl.BlockSpec((BR, D_HID), lambda i: (i, 0)),
            pl.BlockSpec((BR, 16), lambda i: (i, 0)),
            pl.BlockSpec((BR, 16), lambda i: (i, 0)),
            pl.BlockSpec((1, D_HID), lambda i: (0, 0)),
            pl.BlockSpec((D_HID, d_out), lambda i: (0, 0)),
        ],
        out_specs=pl.BlockSpec((BR, d_out), lambda i: (i, 0)),
        out_shape=jax.ShapeDtypeStruct((N_PAD, d_out), jnp.float32),
    )(p0, p1, nd, ns, b, w)


def _stage4_body(p0_ref, p1_ref, nd_ref, b_ref, o_ref):
    o_ref[...] = (p0_ref[...] + p1_ref[...]) * nd_ref[:, 0:1] + b_ref[...]


def _stage4(p0, p1, nd, b):
    return pl.pallas_call(
        _stage4_body,
        grid=(GRID,),
        in_specs=[
            pl.BlockSpec((BR, 16), lambda i: (i, 0)),
            pl.BlockSpec((BR, 16), lambda i: (i, 0)),
            pl.BlockSpec((BR, 16), lambda i: (i, 0)),
            pl.BlockSpec((1, 16), lambda i: (0, 0)),
        ],
        out_specs=pl.BlockSpec((BR, 16), lambda i: (i, 0)),
        out_shape=jax.ShapeDtypeStruct((N_PAD, 16), jnp.float32),
    )(p0, p1, nd, b)


def kernel(feature, edge_index, W1, b1, W2, b2, W3, b3):
    src = edge_index[0]
    dst = edge_index[1]
    fill = jnp.full((E_PAD - E,), N, jnp.int32)
    srcp = jnp.concatenate([src, fill]).reshape(E_PAD // CE, CE)
    dstp = jnp.concatenate([dst, fill]).reshape(E_PAD // CE, CE)
    xpad = jnp.zeros((N_PAD, D_IN), jnp.float32).at[:N].set(feature)
    ones16 = jnp.ones((CE, 16), jnp.float32)
    z16 = jnp.zeros((ZR, 16), jnp.float32)
    z128 = jnp.zeros((ZR, D_HID), jnp.float32)
    W3p = jnp.zeros((D_HID, 16), jnp.float32).at[:, :2].set(W3)
    b3p = jnp.zeros((1, 16), jnp.float32).at[0, :2].set(b3)

    do_p, di_p = _deg_kernel(srcp, dstp, ones16, z16)
    ns, nd = _norms(do_p, di_p)

    g1 = _stage1(xpad, W1, ns)
    p1 = _prop128(g1, srcp, dstp, z128)
    g2 = _stage_mid(p1[0], p1[1], nd, ns, b1.reshape(1, -1), W2, D_HID)
    p2 = _prop128(g2, srcp, dstp, z128)
    g3 = _stage_mid(p2[0], p2[1], nd, ns, b2.reshape(1, -1), W3p, 16)
    p3 = _prop16(g3, srcp, dstp, z16)
    outp = _stage4(p3[0], p3[1], nd, b3p)
    return outp[:N, :2]


# speedup vs baseline: 4.8927x; 1.2992x over previous
import functools

import jax
import jax.numpy as jnp
from jax import lax
from jax.experimental import pallas as pl
from jax.experimental.pallas import tpu as pltpu
from jax.experimental.pallas import tpu_sc as plsc

N = 10000
E = 160000
D_IN = 256
D_HID = 128
DH = 64

NC = 2
NS = 16
NW = NC * NS
N_PAD = 10240
CE = 128
E_PAD = 163840
CHUNKS_W = E_PAD // (NW * CE)
CHUNKS_T = E_PAD // (NS * CE)
ROWS_T = N_PAD // NS
ZR = 64

NB = 2
NBATCH_W = CHUNKS_W // NB
NBATCH_T = CHUNKS_T // NB
NPAIR_T = NBATCH_T // 2

BR = 512
GRID = N_PAD // BR


def _mesh():
    return plsc.VectorSubcoreMesh(core_axis_name="c", subcore_axis_name="s")


@functools.partial(
    pl.kernel,
    out_type=jax.ShapeDtypeStruct((NC, N_PAD, 16), jnp.float32),
    mesh=_mesh(),
    scratch_types=[
        pltpu.VMEM((CHUNKS_W, CE), jnp.int32),
        pltpu.VMEM((CHUNKS_W, CE), jnp.int32),
        pltpu.VMEM((CE, 16), jnp.float32),
        pltpu.VMEM((CE, 16), jnp.float32),
        pltpu.VMEM((ZR, 16), jnp.float32),
        pltpu.VMEM_SHARED((N_PAD, 16), jnp.float32),
    ],
    name="sc_degrees",
    compiler_params=pltpu.CompilerParams(use_tc_tiling_on_sc=False),
)
def _deg_kernel(src_hbm, dst_hbm, ones_src_hbm, ones_dst_hbm, zeros_hbm, deg_hbm,
                srcv, dstv, ones_s, ones_d, zeros_v, acc):
    cid = lax.axis_index("c")
    sid = lax.axis_index("s")
    wid = cid * NS + sid
    pltpu.sync_copy(ones_src_hbm, ones_s)
    pltpu.sync_copy(ones_dst_hbm, ones_d)
    pltpu.sync_copy(zeros_hbm, zeros_v)

    def zbody(i, _):
        pltpu.sync_copy(zeros_v, acc.at[pl.ds(sid * ROWS_T + i * ZR, ZR)])
        return 0
    lax.fori_loop(0, ROWS_T // ZR, zbody, 0)

    pltpu.sync_copy(src_hbm.at[pl.ds(wid * CHUNKS_W, CHUNKS_W)], srcv)
    pltpu.sync_copy(dst_hbm.at[pl.ds(wid * CHUNKS_W, CHUNKS_W)], dstv)
    plsc.subcore_barrier()

    def body(k, _):
        pltpu.sync_copy(ones_s, acc.at[srcv.at[k]], add=True)
        pltpu.sync_copy(ones_d, acc.at[dstv.at[k]], add=True)
        return 0
    lax.fori_loop(0, CHUNKS_W, body, 0)
    plsc.subcore_barrier()

    r0 = sid * ROWS_T
    pltpu.sync_copy(acc.at[pl.ds(r0, ROWS_T)], deg_hbm.at[cid, pl.ds(r0, ROWS_T)])


@functools.partial(
    pl.kernel,
    out_type=jax.ShapeDtypeStruct((NC, N_PAD, DH), jnp.float32),
    mesh=_mesh(),
    scratch_types=[
        pltpu.VMEM((CHUNKS_T, CE), jnp.int32),
        pltpu.VMEM((CHUNKS_T, CE), jnp.int32),
        pltpu.VMEM((NB, CE, DH), jnp.float32),
        pltpu.VMEM((NB, CE, DH), jnp.float32),
        pltpu.VMEM((ZR, DH), jnp.float32),
        pltpu.SemaphoreType.DMA,
        pltpu.SemaphoreType.DMA,
        pltpu.SemaphoreType.DMA,
        pltpu.SemaphoreType.DMA,
        pltpu.VMEM_SHARED((N_PAD, DH), jnp.float32),
    ],
    name="sc_prop_128",
    compiler_params=pltpu.CompilerParams(use_tc_tiling_on_sc=False),
)
def _prop128(tbl_hbm, src_hbm, dst_hbm, zeros_hbm, out_hbm,
             srcv, dstv, rows_a, rows_b, zeros_v,
             gsem_a, gsem_b, ssem_a, ssem_b, acc):
    cid = lax.axis_index("c")
    sid = lax.axis_index("s")
    tbl = tbl_hbm.at[cid]
    pltpu.sync_copy(zeros_hbm, zeros_v)

    def zbody(i, _):
        pltpu.sync_copy(zeros_v, acc.at[pl.ds(sid * ROWS_T + i * ZR, ZR)])
        return 0
    lax.fori_loop(0, ROWS_T // ZR, zbody, 0)

    pltpu.sync_copy(src_hbm.at[pl.ds(sid * CHUNKS_T, CHUNKS_T)], srcv)
    pltpu.sync_copy(dst_hbm.at[pl.ds(sid * CHUNKS_T, CHUNKS_T)], dstv)
    plsc.subcore_barrier()

    def fire_g(rows, sem, t):
        for b in range(NB):
            pltpu.async_copy(tbl.at[srcv.at[t * NB + b]], rows.at[b], sem)

    def drain_g(rows, sem, t):
        for b in range(NB):
            pltpu.make_async_copy(
                tbl.at[srcv.at[t * NB + b]], rows.at[b], sem).wait()

    def fire_s(rows, sem, t):
        for b in range(NB):
            pltpu.async_copy(rows.at[b], acc.at[dstv.at[t * NB + b]], sem,
                             add=True)

    def drain_s(rows, sem, t):
        for b in range(NB):
            pltpu.make_async_copy(
                rows.at[b], acc.at[dstv.at[t * NB + b]], sem).wait()

    fire_g(rows_a, gsem_a, 0)

    def body(h, _):
        ta = 2 * h
        tb = 2 * h + 1
        drain_g(rows_a, gsem_a, ta)
        fire_s(rows_a, ssem_a, ta)

        @pl.when(h > 0)
        def _():
            drain_s(rows_b, ssem_b, tb - 2)
        fire_g(rows_b, gsem_b, tb)
        drain_g(rows_b, gsem_b, tb)
        fire_s(rows_b, ssem_b, tb)

        drain_s(rows_a, ssem_a, ta)

        @pl.when(h < NPAIR_T - 1)
        def _():
            fire_g(rows_a, gsem_a, ta + 2)
        return 0
    lax.fori_loop(0, NPAIR_T, body, 0)
    drain_s(rows_b, ssem_b, 2 * NPAIR_T - 1)
    plsc.subcore_barrier()

    r0 = sid * ROWS_T
    pltpu.sync_copy(acc.at[pl.ds(r0, ROWS_T)],
                    out_hbm.at[cid, pl.ds(r0, ROWS_T)])


@functools.partial(
    pl.kernel,
    out_type=jax.ShapeDtypeStruct((NC, N_PAD, 16), jnp.float32),
    mesh=_mesh(),
    scratch_types=[
        pltpu.VMEM((CHUNKS_W, CE), jnp.int32),
        pltpu.VMEM((CHUNKS_W, CE), jnp.int32),
        pltpu.VMEM((NB, CE, 16), jnp.float32),
        pltpu.VMEM((ZR, 16), jnp.float32),
        pltpu.SemaphoreType.DMA,
        pltpu.VMEM_SHARED((N_PAD, 16), jnp.float32),
    ],
    name="sc_prop_16",
    compiler_params=pltpu.CompilerParams(use_tc_tiling_on_sc=False),
)
def _prop16(tbl_hbm, src_hbm, dst_hbm, zeros_hbm, out_hbm,
            srcv, dstv, rows_v, zeros_v, gsem, acc):
    cid = lax.axis_index("c")
    sid = lax.axis_index("s")
    wid = cid * NS + sid
    pltpu.sync_copy(zeros_hbm, zeros_v)

    def zbody(i, _):
        pltpu.sync_copy(zeros_v, acc.at[pl.ds(sid * ROWS_T + i * ZR, ZR)])
        return 0
    lax.fori_loop(0, ROWS_T // ZR, zbody, 0)

    pltpu.sync_copy(src_hbm.at[pl.ds(wid * CHUNKS_W, CHUNKS_W)], srcv)
    pltpu.sync_copy(dst_hbm.at[pl.ds(wid * CHUNKS_W, CHUNKS_W)], dstv)
    plsc.subcore_barrier()

    def body(g, _):
        for b in range(NB):
            pltpu.async_copy(tbl_hbm.at[srcv.at[g * NB + b]],
                             rows_v.at[b], gsem)
        for b in range(NB):
            pltpu.make_async_copy(tbl_hbm.at[srcv.at[g * NB + b]],
                                  rows_v.at[b], gsem).wait()
            pltpu.sync_copy(rows_v.at[b], acc.at[dstv.at[g * NB + b]],
                            add=True)
        return 0
    lax.fori_loop(0, NBATCH_W, body, 0)
    plsc.subcore_barrier()

    r0 = sid * ROWS_T
    pltpu.sync_copy(acc.at[pl.ds(r0, ROWS_T)], out_hbm.at[cid, pl.ds(r0, ROWS_T)])


def _norms_body(d0, d1, ns_ref, nd_ref):
    deg = d0[...] + d1[...]
    ns_ref[...] = lax.rsqrt(jnp.maximum(deg[:, 0:1], 1.0)) * jnp.ones(
        (1, 16), jnp.float32)
    nd_ref[...] = lax.rsqrt(jnp.maximum(deg[:, 8:9], 1.0)) * jnp.ones(
        (1, 16), jnp.float32)


def _norms(deg_p):
    return pl.pallas_call(
        _norms_body,
        out_shape=(jax.ShapeDtypeStruct((N_PAD, 16), jnp.float32),
                   jax.ShapeDtypeStruct((N_PAD, 16), jnp.float32)),
    )(deg_p[0], deg_p[1])


def _split_halves(o_ref, res):
    o_ref[0, :, :] = res[:, :DH]
    o_ref[1, :, :] = res[:, DH:]


def _stage1_body(x_ref, w_ref, ns_ref, o_ref):
    res = jnp.dot(x_ref[...], w_ref[...],
                  preferred_element_type=jnp.float32) * ns_ref[:, 0:1]
    _split_halves(o_ref, res)


def _stage1(x, w, ns):
    return pl.pallas_call(
        _stage1_body,
        grid=(GRID,),
        in_specs=[
            pl.BlockSpec((BR, D_IN), lambda i: (i, 0)),
            pl.BlockSpec((D_IN, D_HID), lambda i: (0, 0)),
            pl.BlockSpec((BR, 16), lambda i: (i, 0)),
        ],
        out_specs=pl.BlockSpec((NC, BR, DH), lambda i: (0, i, 0)),
        out_shape=jax.ShapeDtypeStruct((NC, N_PAD, DH), jnp.float32),
    )(x, w, ns)


def _stage2_body(p_ref, nd_ref, ns_ref, b_ref, w_ref, o_ref):
    p = jnp.concatenate([p_ref[0], p_ref[1]], axis=1)
    h = p * nd_ref[:, 0:1] + b_ref[...]
    res = jnp.dot(h, w_ref[...],
                  preferred_element_type=jnp.float32) * ns_ref[:, 0:1]
    _split_halves(o_ref, res)


def _stage2(p, nd, ns, b, w):
    return pl.pallas_call(
        _stage2_body,
        grid=(GRID,),
        in_specs=[
            pl.BlockSpec((NC, BR, DH), lambda i: (0, i, 0)),
            pl.BlockSpec((BR, 16), lambda i: (i, 0)),
            pl.BlockSpec((BR, 16), lambda i: (i, 0)),
            pl.BlockSpec((1, D_HID), lambda i: (0, 0)),
            pl.BlockSpec((D_HID, D_HID), lambda i: (0, 0)),
        ],
        out_specs=pl.BlockSpec((NC, BR, DH), lambda i: (0, i, 0)),
        out_shape=jax.ShapeDtypeStruct((NC, N_PAD, DH), jnp.float32),
    )(p, nd, ns, b, w)


def _stage3_body(p_ref, nd_ref, ns_ref, b_ref, w_ref, o_ref):
    p = jnp.concatenate([p_ref[0], p_ref[1]], axis=1)
    h = p * nd_ref[:, 0:1] + b_ref[...]
    o_ref[...] = jnp.dot(h, w_ref[...],
                         preferred_element_type=jnp.float32) * ns_ref[:, 0:1]


def _stage3(p, nd, ns, b, w):
    return pl.pallas_call(
        _stage3_body,
        grid=(GRID,),
        in_specs=[
            pl.BlockSpec((NC, BR, DH), lambda i: (0, i, 0)),
            pl.BlockSpec((BR, 16), lambda i: (i, 0)),
            pl.BlockSpec((BR, 16), lambda i: (i, 0)),
            pl.BlockSpec((1, D_HID), lambda i: (0, 0)),
            pl.BlockSpec((D_HID, 16), lambda i: (0, 0)),
        ],
        out_specs=pl.BlockSpec((BR, 16), lambda i: (i, 0)),
        out_shape=jax.ShapeDtypeStruct((N_PAD, 16), jnp.float32),
    )(p, nd, ns, b, w)


def _stage4_body(p0_ref, p1_ref, nd_ref, b_ref, o_ref):
    o_ref[...] = (p0_ref[...] + p1_ref[...]) * nd_ref[:, 0:1] + b_ref[...]


def _stage4(p0, p1, nd, b):
    return pl.pallas_call(
        _stage4_body,
        grid=(GRID,),
        in_specs=[
            pl.BlockSpec((BR, 16), lambda i: (i, 0)),
            pl.BlockSpec((BR, 16), lambda i: (i, 0)),
            pl.BlockSpec((BR, 16), lambda i: (i, 0)),
            pl.BlockSpec((1, 16), lambda i: (0, 0)),
        ],
        out_specs=pl.BlockSpec((BR, 16), lambda i: (i, 0)),
        out_shape=jax.ShapeDtypeStruct((N_PAD, 16), jnp.float32),
    )(p0, p1, nd, b)


def kernel(feature, edge_index, W1, b1, W2, b2, W3, b3):
    src = edge_index[0]
    dst = edge_index[1]
    fill = jnp.full((E_PAD - E,), N, jnp.int32)
    srcp = jnp.concatenate([src, fill]).reshape(E_PAD // CE, CE)
    dstp = jnp.concatenate([dst, fill]).reshape(E_PAD // CE, CE)
    xpad = jnp.zeros((N_PAD, D_IN), jnp.float32).at[:N].set(feature)
    col = jnp.arange(16)
    ones_src = jnp.broadcast_to((col < 8).astype(jnp.float32), (CE, 16))
    ones_dst = jnp.broadcast_to((col >= 8).astype(jnp.float32), (CE, 16))
    z16 = jnp.zeros((ZR, 16), jnp.float32)
    z64 = jnp.zeros((ZR, DH), jnp.float32)
    W3p = jnp.zeros((D_HID, 16), jnp.float32).at[:, :2].set(W3)
    b3p = jnp.zeros((1, 16), jnp.float32).at[0, :2].set(b3)

    deg_p = _deg_kernel(srcp, dstp, ones_src, ones_dst, z16)
    ns, nd = _norms(deg_p)

    g1 = _stage1(xpad, W1, ns)
    p1 = _prop128(g1, srcp, dstp, z64)
    g2 = _stage2(p1, nd, ns, b1.reshape(1, -1), W2)
    p2 = _prop128(g2, srcp, dstp, z64)
    g3 = _stage3(p2, nd, ns, b2.reshape(1, -1), W3p)
    p3 = _prop16(g3, srcp, dstp, z16)
    outp = _stage4(p3[0], p3[1], nd, b3p)
    return outp[:N, :2]


# speedup vs baseline: 4.9872x; 1.0193x over previous
import functools

import jax
import jax.numpy as jnp
from jax import lax
from jax.experimental import pallas as pl
from jax.experimental.pallas import tpu as pltpu
from jax.experimental.pallas import tpu_sc as plsc

N = 10000
E = 160000
D_IN = 256
D_HID = 128
DH = 64

NC = 2
NS = 16
NW = NC * NS
N_PAD = 10240
CE = 128
E_PAD = 163840
CHUNKS_W = E_PAD // (NW * CE)
CHUNKS_T = E_PAD // (NS * CE)
ROWS_T = N_PAD // NS
ZR = 64

NB = 2
NB128 = 4
NBATCH_W = CHUNKS_W // NB
NBATCH_T = CHUNKS_T // NB128
NPAIR_T = NBATCH_T // 2

BR = 512
GRID = N_PAD // BR


def _mesh():
    return plsc.VectorSubcoreMesh(core_axis_name="c", subcore_axis_name="s")


@functools.partial(
    pl.kernel,
    out_type=jax.ShapeDtypeStruct((NC, N_PAD, 16), jnp.float32),
    mesh=_mesh(),
    scratch_types=[
        pltpu.VMEM((CHUNKS_W, CE), jnp.int32),
        pltpu.VMEM((CHUNKS_W, CE), jnp.int32),
        pltpu.VMEM((CE, 16), jnp.float32),
        pltpu.VMEM((CE, 16), jnp.float32),
        pltpu.VMEM((ZR, 16), jnp.float32),
        pltpu.VMEM_SHARED((N_PAD, 16), jnp.float32),
    ],
    name="sc_degrees",
    compiler_params=pltpu.CompilerParams(use_tc_tiling_on_sc=False),
)
def _deg_kernel(src_hbm, dst_hbm, ones_src_hbm, ones_dst_hbm, zeros_hbm, deg_hbm,
                srcv, dstv, ones_s, ones_d, zeros_v, acc):
    cid = lax.axis_index("c")
    sid = lax.axis_index("s")
    wid = cid * NS + sid
    pltpu.sync_copy(ones_src_hbm, ones_s)
    pltpu.sync_copy(ones_dst_hbm, ones_d)
    pltpu.sync_copy(zeros_hbm, zeros_v)

    def zbody(i, _):
        pltpu.sync_copy(zeros_v, acc.at[pl.ds(sid * ROWS_T + i * ZR, ZR)])
        return 0
    lax.fori_loop(0, ROWS_T // ZR, zbody, 0)

    pltpu.sync_copy(src_hbm.at[pl.ds(wid * CHUNKS_W, CHUNKS_W)], srcv)
    pltpu.sync_copy(dst_hbm.at[pl.ds(wid * CHUNKS_W, CHUNKS_W)], dstv)
    plsc.subcore_barrier()

    def body(k, _):
        pltpu.sync_copy(ones_s, acc.at[srcv.at[k]], add=True)
        pltpu.sync_copy(ones_d, acc.at[dstv.at[k]], add=True)
        return 0
    lax.fori_loop(0, CHUNKS_W, body, 0)
    plsc.subcore_barrier()

    r0 = sid * ROWS_T
    pltpu.sync_copy(acc.at[pl.ds(r0, ROWS_T)], deg_hbm.at[cid, pl.ds(r0, ROWS_T)])


@functools.partial(
    pl.kernel,
    out_type=jax.ShapeDtypeStruct((NC, N_PAD, DH), jnp.float32),
    mesh=_mesh(),
    scratch_types=[
        pltpu.VMEM((CHUNKS_T, CE), jnp.int32),
        pltpu.VMEM((CHUNKS_T, CE), jnp.int32),
        pltpu.VMEM((NB128, CE, DH), jnp.float32),
        pltpu.VMEM((NB128, CE, DH), jnp.float32),
        pltpu.VMEM((ZR, DH), jnp.float32),
        pltpu.SemaphoreType.DMA,
        pltpu.SemaphoreType.DMA,
        pltpu.SemaphoreType.DMA,
        pltpu.SemaphoreType.DMA,
        pltpu.VMEM_SHARED((N_PAD, DH), jnp.float32),
    ],
    name="sc_prop_128",
    compiler_params=pltpu.CompilerParams(use_tc_tiling_on_sc=False),
)
def _prop128(tbl_hbm, src_hbm, dst_hbm, zeros_hbm, out_hbm,
             srcv, dstv, rows_a, rows_b, zeros_v,
             gsem_a, gsem_b, ssem_a, ssem_b, acc):
    cid = lax.axis_index("c")
    sid = lax.axis_index("s")
    tbl = tbl_hbm.at[cid]
    pltpu.sync_copy(zeros_hbm, zeros_v)

    def zbody(i, _):
        pltpu.sync_copy(zeros_v, acc.at[pl.ds(sid * ROWS_T + i * ZR, ZR)])
        return 0
    lax.fori_loop(0, ROWS_T // ZR, zbody, 0)

    pltpu.sync_copy(src_hbm.at[pl.ds(sid * CHUNKS_T, CHUNKS_T)], srcv)
    pltpu.sync_copy(dst_hbm.at[pl.ds(sid * CHUNKS_T, CHUNKS_T)], dstv)
    plsc.subcore_barrier()

    def fire_g(rows, sem, t):
        for b in range(NB128):
            pltpu.async_copy(tbl.at[srcv.at[t * NB128 + b]], rows.at[b], sem)

    def drain_g(rows, sem, t):
        for b in range(NB128):
            pltpu.make_async_copy(
                tbl.at[srcv.at[t * NB128 + b]], rows.at[b], sem).wait()

    def fire_s(rows, sem, t):
        for b in range(NB128):
            pltpu.async_copy(rows.at[b], acc.at[dstv.at[t * NB128 + b]], sem,
                             add=True)

    def drain_s(rows, sem, t):
        for b in range(NB128):
            pltpu.make_async_copy(
                rows.at[b], acc.at[dstv.at[t * NB128 + b]], sem).wait()

    fire_g(rows_a, gsem_a, 0)

    def body(h, _):
        ta = 2 * h
        tb = 2 * h + 1
        drain_g(rows_a, gsem_a, ta)
        fire_s(rows_a, ssem_a, ta)

        @pl.when(h > 0)
        def _():
            drain_s(rows_b, ssem_b, tb - 2)
        fire_g(rows_b, gsem_b, tb)
        drain_g(rows_b, gsem_b, tb)
        fire_s(rows_b, ssem_b, tb)

        drain_s(rows_a, ssem_a, ta)

        @pl.when(h < NPAIR_T - 1)
        def _():
            fire_g(rows_a, gsem_a, ta + 2)
        return 0
    lax.fori_loop(0, NPAIR_T, body, 0)
    drain_s(rows_b, ssem_b, 2 * NPAIR_T - 1)
    plsc.subcore_barrier()

    r0 = sid * ROWS_T
    pltpu.sync_copy(acc.at[pl.ds(r0, ROWS_T)],
                    out_hbm.at[cid, pl.ds(r0, ROWS_T)])


@functools.partial(
    pl.kernel,
    out_type=jax.ShapeDtypeStruct((NC, N_PAD, 16), jnp.float32),
    mesh=_mesh(),
    scratch_types=[
        pltpu.VMEM((CHUNKS_W, CE), jnp.int32),
        pltpu.VMEM((CHUNKS_W, CE), jnp.int32),
        pltpu.VMEM((NB, CE, 16), jnp.float32),
        pltpu.VMEM((ZR, 16), jnp.float32),
        pltpu.SemaphoreType.DMA,
        pltpu.VMEM_SHARED((N_PAD, 16), jnp.float32),
    ],
    name="sc_prop_16",
    compiler_params=pltpu.CompilerParams(use_tc_tiling_on_sc=False),
)
def _prop16(tbl_hbm, src_hbm, dst_hbm, zeros_hbm, out_hbm,
            srcv, dstv, rows_v, zeros_v, gsem, acc):
    cid = lax.axis_index("c")
    sid = lax.axis_index("s")
    wid = cid * NS + sid
    pltpu.sync_copy(zeros_hbm, zeros_v)

    def zbody(i, _):
        pltpu.sync_copy(zeros_v, acc.at[pl.ds(sid * ROWS_T + i * ZR, ZR)])
        return 0
    lax.fori_loop(0, ROWS_T // ZR, zbody, 0)

    pltpu.sync_copy(src_hbm.at[pl.ds(wid * CHUNKS_W, CHUNKS_W)], srcv)
    pltpu.sync_copy(dst_hbm.at[pl.ds(wid * CHUNKS_W, CHUNKS_W)], dstv)
    plsc.subcore_barrier()

    def body(g, _):
        for b in range(NB):
            pltpu.async_copy(tbl_hbm.at[srcv.at[g * NB + b]],
                             rows_v.at[b], gsem)
        for b in range(NB):
            pltpu.make_async_copy(tbl_hbm.at[srcv.at[g * NB + b]],
                                  rows_v.at[b], gsem).wait()
            pltpu.sync_copy(rows_v.at[b], acc.at[dstv.at[g * NB + b]],
                            add=True)
        return 0
    lax.fori_loop(0, NBATCH_W, body, 0)
    plsc.subcore_barrier()

    r0 = sid * ROWS_T
    pltpu.sync_copy(acc.at[pl.ds(r0, ROWS_T)], out_hbm.at[cid, pl.ds(r0, ROWS_T)])


def _norms_body(d0, d1, ns_ref, nd_ref):
    deg = d0[...] + d1[...]
    ns_ref[...] = lax.rsqrt(jnp.maximum(deg[:, 0:1], 1.0)) * jnp.ones(
        (1, 16), jnp.float32)
    nd_ref[...] = lax.rsqrt(jnp.maximum(deg[:, 8:9], 1.0)) * jnp.ones(
        (1, 16), jnp.float32)


def _norms(deg_p):
    return pl.pallas_call(
        _norms_body,
        out_shape=(jax.ShapeDtypeStruct((N_PAD, 16), jnp.float32),
                   jax.ShapeDtypeStruct((N_PAD, 16), jnp.float32)),
    )(deg_p[0], deg_p[1])


def _split_halves(o_ref, res):
    o_ref[0, :, :] = res[:, :DH]
    o_ref[1, :, :] = res[:, DH:]


def _stage1_body(x_ref, w_ref, ns_ref, o_ref):
    res = jnp.dot(x_ref[...], w_ref[...],
                  preferred_element_type=jnp.float32) * ns_ref[:, 0:1]
    _split_halves(o_ref, res)


def _stage1(x, w, ns):
    return pl.pallas_call(
        _stage1_body,
        grid=(GRID,),
        in_specs=[
            pl.BlockSpec((BR, D_IN), lambda i: (i, 0)),
            pl.BlockSpec((D_IN, D_HID), lambda i: (0, 0)),
            pl.BlockSpec((BR, 16), lambda i: (i, 0)),
        ],
        out_specs=pl.BlockSpec((NC, BR, DH), lambda i: (0, i, 0)),
        out_shape=jax.ShapeDtypeStruct((NC, N_PAD, DH), jnp.float32),
    )(x, w, ns)


def _stage2_body(p_ref, nd_ref, ns_ref, b_ref, w_ref, o_ref):
    p = jnp.concatenate([p_ref[0], p_ref[1]], axis=1)
    h = p * nd_ref[:, 0:1] + b_ref[...]
    res = jnp.dot(h, w_ref[...],
                  preferred_element_type=jnp.float32) * ns_ref[:, 0:1]
    _split_halves(o_ref, res)


def _stage2(p, nd, ns, b, w):
    return pl.pallas_call(
        _stage2_body,
        grid=(GRID,),
        in_specs=[
            pl.BlockSpec((NC, BR, DH), lambda i: (0, i, 0)),
            pl.BlockSpec((BR, 16), lambda i: (i, 0)),
            pl.BlockSpec((BR, 16), lambda i: (i, 0)),
            pl.BlockSpec((1, D_HID), lambda i: (0, 0)),
            pl.BlockSpec((D_HID, D_HID), lambda i: (0, 0)),
        ],
        out_specs=pl.BlockSpec((NC, BR, DH), lambda i: (0, i, 0)),
        out_shape=jax.ShapeDtypeStruct((NC, N_PAD, DH), jnp.float32),
    )(p, nd, ns, b, w)


def _stage3_body(p_ref, nd_ref, ns_ref, b_ref, w_ref, o_ref):
    p = jnp.concatenate([p_ref[0], p_ref[1]], axis=1)
    h = p * nd_ref[:, 0:1] + b_ref[...]
    o_ref[...] = jnp.dot(h, w_ref[...],
                         preferred_element_type=jnp.float32) * ns_ref[:, 0:1]


def _stage3(p, nd, ns, b, w):
    return pl.pallas_call(
        _stage3_body,
        grid=(GRID,),
        in_specs=[
            pl.BlockSpec((NC, BR, DH), lambda i: (0, i, 0)),
            pl.BlockSpec((BR, 16), lambda i: (i, 0)),
            pl.BlockSpec((BR, 16), lambda i: (i, 0)),
            pl.BlockSpec((1, D_HID), lambda i: (0, 0)),
            pl.BlockSpec((D_HID, 16), lambda i: (0, 0)),
        ],
        out_specs=pl.BlockSpec((BR, 16), lambda i: (i, 0)),
        out_shape=jax.ShapeDtypeStruct((N_PAD, 16), jnp.float32),
    )(p, nd, ns, b, w)


def _stage4_body(p0_ref, p1_ref, nd_ref, b_ref, o_ref):
    o_ref[...] = (p0_ref[...] + p1_ref[...]) * nd_ref[:, 0:1] + b_ref[...]


def _stage4(p0, p1, nd, b):
    return pl.pallas_call(
        _stage4_body,
        grid=(GRID,),
        in_specs=[
            pl.BlockSpec((BR, 16), lambda i: (i, 0)),
            pl.BlockSpec((BR, 16), lambda i: (i, 0)),
            pl.BlockSpec((BR, 16), lambda i: (i, 0)),
            pl.BlockSpec((1, 16), lambda i: (0, 0)),
        ],
        out_specs=pl.BlockSpec((BR, 16), lambda i: (i, 0)),
        out_shape=jax.ShapeDtypeStruct((N_PAD, 16), jnp.float32),
    )(p0, p1, nd, b)


def kernel(feature, edge_index, W1, b1, W2, b2, W3, b3):
    src = edge_index[0]
    dst = edge_index[1]
    fill = jnp.full((E_PAD - E,), N, jnp.int32)
    srcp = jnp.concatenate([src, fill]).reshape(E_PAD // CE, CE)
    dstp = jnp.concatenate([dst, fill]).reshape(E_PAD // CE, CE)
    xpad = jnp.zeros((N_PAD, D_IN), jnp.float32).at[:N].set(feature)
    col = jnp.arange(16)
    ones_src = jnp.broadcast_to((col < 8).astype(jnp.float32), (CE, 16))
    ones_dst = jnp.broadcast_to((col >= 8).astype(jnp.float32), (CE, 16))
    z16 = jnp.zeros((ZR, 16), jnp.float32)
    z64 = jnp.zeros((ZR, DH), jnp.float32)
    W3p = jnp.zeros((D_HID, 16), jnp.float32).at[:, :2].set(W3)
    b3p = jnp.zeros((1, 16), jnp.float32).at[0, :2].set(b3)

    deg_p = _deg_kernel(srcp, dstp, ones_src, ones_dst, z16)
    ns, nd = _norms(deg_p)

    g1 = _stage1(xpad, W1, ns)
    p1 = _prop128(g1, srcp, dstp, z64)
    g2 = _stage2(p1, nd, ns, b1.reshape(1, -1), W2)
    p2 = _prop128(g2, srcp, dstp, z64)
    g3 = _stage3(p2, nd, ns, b2.reshape(1, -1), W3p)
    p3 = _prop16(g3, srcp, dstp, z16)
    outp = _stage4(p3[0], p3[1], nd, b3p)
    return outp[:N, :2]


# speedup vs baseline: 5.2228x; 1.0472x over previous
import functools

import jax
import jax.numpy as jnp
from jax import lax
from jax.experimental import pallas as pl
from jax.experimental.pallas import tpu as pltpu
from jax.experimental.pallas import tpu_sc as plsc

N = 10000
E = 160000
D_IN = 256
D_HID = 128
DH = 64

NC = 2
NS = 16
NW = NC * NS
N_PAD = 10240
CE = 128
E_PAD = 163840
CHUNKS_W = E_PAD // (NW * CE)
CHUNKS_T = E_PAD // (NS * CE)
ROWS_T = N_PAD // NS
ZR = 64

NB = 2
NB128 = 4
NBATCH_W = CHUNKS_W // NB
NBATCH_T = CHUNKS_T // NB128
NPAIR_T = NBATCH_T // 2

BR = 512
GRID = N_PAD // BR


def _mesh():
    return plsc.VectorSubcoreMesh(core_axis_name="c", subcore_axis_name="s")


@functools.partial(
    pl.kernel,
    out_type=jax.ShapeDtypeStruct((NC, N_PAD, 16), jnp.float32),
    mesh=_mesh(),
    scratch_types=[
        pltpu.VMEM((CHUNKS_W, CE), jnp.int32),
        pltpu.VMEM((CHUNKS_W, CE), jnp.int32),
        pltpu.VMEM((CE, 16), jnp.float32),
        pltpu.VMEM((CE, 16), jnp.float32),
        pltpu.VMEM((ZR, 16), jnp.float32),
        pltpu.VMEM_SHARED((N_PAD, 16), jnp.float32),
    ],
    name="sc_degrees",
    compiler_params=pltpu.CompilerParams(use_tc_tiling_on_sc=False),
)
def _deg_kernel(src_hbm, dst_hbm, ones_src_hbm, ones_dst_hbm, zeros_hbm, deg_hbm,
                srcv, dstv, ones_s, ones_d, zeros_v, acc):
    cid = lax.axis_index("c")
    sid = lax.axis_index("s")
    wid = cid * NS + sid
    pltpu.sync_copy(ones_src_hbm, ones_s)
    pltpu.sync_copy(ones_dst_hbm, ones_d)
    pltpu.sync_copy(zeros_hbm, zeros_v)

    def zbody(i, _):
        pltpu.sync_copy(zeros_v, acc.at[pl.ds(sid * ROWS_T + i * ZR, ZR)])
        return 0
    lax.fori_loop(0, ROWS_T // ZR, zbody, 0)

    pltpu.sync_copy(src_hbm.at[pl.ds(wid * CHUNKS_W, CHUNKS_W)], srcv)
    pltpu.sync_copy(dst_hbm.at[pl.ds(wid * CHUNKS_W, CHUNKS_W)], dstv)
    plsc.subcore_barrier()

    def body(k, _):
        pltpu.sync_copy(ones_s, acc.at[srcv.at[k]], add=True)
        pltpu.sync_copy(ones_d, acc.at[dstv.at[k]], add=True)
        return 0
    lax.fori_loop(0, CHUNKS_W, body, 0)
    plsc.subcore_barrier()

    r0 = sid * ROWS_T
    pltpu.sync_copy(acc.at[pl.ds(r0, ROWS_T)], deg_hbm.at[cid, pl.ds(r0, ROWS_T)])


@functools.partial(
    pl.kernel,
    out_type=jax.ShapeDtypeStruct((NC, N_PAD, DH), jnp.float32),
    mesh=_mesh(),
    scratch_types=[
        pltpu.VMEM((CHUNKS_T, CE), jnp.int32),
        pltpu.VMEM((CHUNKS_T, CE), jnp.int32),
        pltpu.VMEM((NB128, CE, DH), jnp.float32),
        pltpu.VMEM((NB128, CE, DH), jnp.float32),
        pltpu.VMEM((ZR, DH), jnp.float32),
        pltpu.SemaphoreType.DMA,
        pltpu.SemaphoreType.DMA,
        pltpu.SemaphoreType.DMA,
        pltpu.SemaphoreType.DMA,
        pltpu.VMEM_SHARED((N_PAD, DH), jnp.float32),
    ],
    name="sc_prop_128",
    compiler_params=pltpu.CompilerParams(use_tc_tiling_on_sc=False),
)
def _prop128(tbl_hbm, src_hbm, dst_hbm, zeros_hbm, out_hbm,
             srcv, dstv, rows_a, rows_b, zeros_v,
             gsem_a, gsem_b, ssem_a, ssem_b, acc):
    cid = lax.axis_index("c")
    sid = lax.axis_index("s")
    tbl = tbl_hbm.at[cid]
    pltpu.sync_copy(zeros_hbm, zeros_v)

    def zbody(i, _):
        pltpu.sync_copy(zeros_v, acc.at[pl.ds(sid * ROWS_T + i * ZR, ZR)])
        return 0
    lax.fori_loop(0, ROWS_T // ZR, zbody, 0)

    pltpu.sync_copy(src_hbm.at[pl.ds(sid * CHUNKS_T, CHUNKS_T)], srcv)
    pltpu.sync_copy(dst_hbm.at[pl.ds(sid * CHUNKS_T, CHUNKS_T)], dstv)
    plsc.subcore_barrier()

    def fire_g(rows, sem, t):
        for b in range(NB128):
            pltpu.async_copy(tbl.at[srcv.at[t * NB128 + b]], rows.at[b], sem)

    def drain_g(rows, sem, t):
        for b in range(NB128):
            pltpu.make_async_copy(
                tbl.at[srcv.at[t * NB128 + b]], rows.at[b], sem).wait()

    def fire_s(rows, sem, t):
        for b in range(NB128):
            pltpu.async_copy(rows.at[b], acc.at[dstv.at[t * NB128 + b]], sem,
                             add=True)

    def drain_s(rows, sem, t):
        for b in range(NB128):
            pltpu.make_async_copy(
                rows.at[b], acc.at[dstv.at[t * NB128 + b]], sem).wait()

    fire_g(rows_a, gsem_a, 0)

    def body(h, _):
        ta = 2 * h
        tb = 2 * h + 1
        drain_g(rows_a, gsem_a, ta)
        fire_s(rows_a, ssem_a, ta)

        @pl.when(h > 0)
        def _():
            drain_s(rows_b, ssem_b, tb - 2)
        fire_g(rows_b, gsem_b, tb)
        drain_g(rows_b, gsem_b, tb)
        fire_s(rows_b, ssem_b, tb)

        drain_s(rows_a, ssem_a, ta)

        @pl.when(h < NPAIR_T - 1)
        def _():
            fire_g(rows_a, gsem_a, ta + 2)
        return 0
    lax.fori_loop(0, NPAIR_T, body, 0)
    drain_s(rows_b, ssem_b, 2 * NPAIR_T - 1)
    plsc.subcore_barrier()

    r0 = sid * ROWS_T
    pltpu.sync_copy(acc.at[pl.ds(r0, ROWS_T)],
                    out_hbm.at[cid, pl.ds(r0, ROWS_T)])


@functools.partial(
    pl.kernel,
    out_type=jax.ShapeDtypeStruct((NC, N_PAD, 16), jnp.float32),
    mesh=_mesh(),
    scratch_types=[
        pltpu.VMEM((CHUNKS_W, CE), jnp.int32),
        pltpu.VMEM((CHUNKS_W, CE), jnp.int32),
        pltpu.VMEM((NB, CE, 16), jnp.float32),
        pltpu.VMEM((ZR, 16), jnp.float32),
        pltpu.SemaphoreType.DMA,
        pltpu.VMEM_SHARED((N_PAD, 16), jnp.float32),
    ],
    name="sc_prop_16",
    compiler_params=pltpu.CompilerParams(use_tc_tiling_on_sc=False),
)
def _prop16(tbl_hbm, src_hbm, dst_hbm, zeros_hbm, out_hbm,
            srcv, dstv, rows_v, zeros_v, gsem, acc):
    cid = lax.axis_index("c")
    sid = lax.axis_index("s")
    wid = cid * NS + sid
    pltpu.sync_copy(zeros_hbm, zeros_v)

    def zbody(i, _):
        pltpu.sync_copy(zeros_v, acc.at[pl.ds(sid * ROWS_T + i * ZR, ZR)])
        return 0
    lax.fori_loop(0, ROWS_T // ZR, zbody, 0)

    pltpu.sync_copy(src_hbm.at[pl.ds(wid * CHUNKS_W, CHUNKS_W)], srcv)
    pltpu.sync_copy(dst_hbm.at[pl.ds(wid * CHUNKS_W, CHUNKS_W)], dstv)
    plsc.subcore_barrier()

    def body(g, _):
        for b in range(NB):
            pltpu.async_copy(tbl_hbm.at[srcv.at[g * NB + b]],
                             rows_v.at[b], gsem)
        for b in range(NB):
            pltpu.make_async_copy(tbl_hbm.at[srcv.at[g * NB + b]],
                                  rows_v.at[b], gsem).wait()
            pltpu.sync_copy(rows_v.at[b], acc.at[dstv.at[g * NB + b]],
                            add=True)
        return 0
    lax.fori_loop(0, NBATCH_W, body, 0)
    plsc.subcore_barrier()

    r0 = sid * ROWS_T
    pltpu.sync_copy(acc.at[pl.ds(r0, ROWS_T)], out_hbm.at[cid, pl.ds(r0, ROWS_T)])


def _split_halves(o_ref, res):
    o_ref[0, :, :] = res[:, :DH]
    o_ref[1, :, :] = res[:, DH:]


def _mm1_body(x_ref, w_ref, o_ref):
    o_ref[...] = jnp.dot(x_ref[...], w_ref[...],
                         preferred_element_type=jnp.float32)


def _mm1(x, w):
    return pl.pallas_call(
        _mm1_body,
        grid=(GRID,),
        in_specs=[
            pl.BlockSpec((BR, D_IN), lambda i: (i, 0)),
            pl.BlockSpec((D_IN, D_HID), lambda i: (0, 0)),
        ],
        out_specs=pl.BlockSpec((BR, D_HID), lambda i: (i, 0)),
        out_shape=jax.ShapeDtypeStruct((N_PAD, D_HID), jnp.float32),
    )(x, w)


def _scale1_body(xw_ref, d0, d1, ns_ref, nd_ref, o_ref):
    deg = d0[...] + d1[...]
    ns = lax.rsqrt(jnp.maximum(deg[:, 0:1], 1.0))
    nd = lax.rsqrt(jnp.maximum(deg[:, 8:9], 1.0))
    ns_ref[...] = ns * jnp.ones((1, 16), jnp.float32)
    nd_ref[...] = nd * jnp.ones((1, 16), jnp.float32)
    _split_halves(o_ref, xw_ref[...] * ns)


def _scale1(xw, deg_p):
    return pl.pallas_call(
        _scale1_body,
        grid=(GRID,),
        in_specs=[
            pl.BlockSpec((BR, D_HID), lambda i: (i, 0)),
            pl.BlockSpec((BR, 16), lambda i: (i, 0)),
            pl.BlockSpec((BR, 16), lambda i: (i, 0)),
        ],
        out_specs=(pl.BlockSpec((BR, 16), lambda i: (i, 0)),
                   pl.BlockSpec((BR, 16), lambda i: (i, 0)),
                   pl.BlockSpec((NC, BR, DH), lambda i: (0, i, 0))),
        out_shape=(jax.ShapeDtypeStruct((N_PAD, 16), jnp.float32),
                   jax.ShapeDtypeStruct((N_PAD, 16), jnp.float32),
                   jax.ShapeDtypeStruct((NC, N_PAD, DH), jnp.float32)),
    )(xw, deg_p[0], deg_p[1])


def _stage2_body(p_ref, nd_ref, ns_ref, b_ref, w_ref, o_ref):
    p = jnp.concatenate([p_ref[0], p_ref[1]], axis=1)
    h = p * nd_ref[:, 0:1] + b_ref[...]
    res = jnp.dot(h, w_ref[...],
                  preferred_element_type=jnp.float32) * ns_ref[:, 0:1]
    _split_halves(o_ref, res)


def _stage2(p, nd, ns, b, w):
    return pl.pallas_call(
        _stage2_body,
        grid=(GRID,),
        in_specs=[
            pl.BlockSpec((NC, BR, DH), lambda i: (0, i, 0)),
            pl.BlockSpec((BR, 16), lambda i: (i, 0)),
            pl.BlockSpec((BR, 16), lambda i: (i, 0)),
            pl.BlockSpec((1, D_HID), lambda i: (0, 0)),
            pl.BlockSpec((D_HID, D_HID), lambda i: (0, 0)),
        ],
        out_specs=pl.BlockSpec((NC, BR, DH), lambda i: (0, i, 0)),
        out_shape=jax.ShapeDtypeStruct((NC, N_PAD, DH), jnp.float32),
    )(p, nd, ns, b, w)


def _stage3_body(p_ref, nd_ref, ns_ref, b_ref, w_ref, o_ref):
    p = jnp.concatenate([p_ref[0], p_ref[1]], axis=1)
    h = p * nd_ref[:, 0:1] + b_ref[...]
    o_ref[...] = jnp.dot(h, w_ref[...],
                         preferred_element_type=jnp.float32) * ns_ref[:, 0:1]


def _stage3(p, nd, ns, b, w):
    return pl.pallas_call(
        _stage3_body,
        grid=(GRID,),
        in_specs=[
            pl.BlockSpec((NC, BR, DH), lambda i: (0, i, 0)),
            pl.BlockSpec((BR, 16), lambda i: (i, 0)),
            pl.BlockSpec((BR, 16), lambda i: (i, 0)),
            pl.BlockSpec((1, D_HID), lambda i: (0, 0)),
            pl.BlockSpec((D_HID, 16), lambda i: (0, 0)),
        ],
        out_specs=pl.BlockSpec((BR, 16), lambda i: (i, 0)),
        out_shape=jax.ShapeDtypeStruct((N_PAD, 16), jnp.float32),
    )(p, nd, ns, b, w)


def _stage4_body(p0_ref, p1_ref, nd_ref, b_ref, o_ref):
    o_ref[...] = (p0_ref[...] + p1_ref[...]) * nd_ref[:, 0:1] + b_ref[...]


def _stage4(p0, p1, nd, b):
    return pl.pallas_call(
        _stage4_body,
        grid=(GRID,),
        in_specs=[
            pl.BlockSpec((BR, 16), lambda i: (i, 0)),
            pl.BlockSpec((BR, 16), lambda i: (i, 0)),
            pl.BlockSpec((BR, 16), lambda i: (i, 0)),
            pl.BlockSpec((1, 16), lambda i: (0, 0)),
        ],
        out_specs=pl.BlockSpec((BR, 16), lambda i: (i, 0)),
        out_shape=jax.ShapeDtypeStruct((N_PAD, 16), jnp.float32),
    )(p0, p1, nd, b)


def kernel(feature, edge_index, W1, b1, W2, b2, W3, b3):
    src = edge_index[0]
    dst = edge_index[1]
    fill = jnp.full((E_PAD - E,), N, jnp.int32)
    srcp = jnp.concatenate([src, fill]).reshape(E_PAD // CE, CE)
    dstp = jnp.concatenate([dst, fill]).reshape(E_PAD // CE, CE)
    xpad = jnp.zeros((N_PAD, D_IN), jnp.float32).at[:N].set(feature)
    col = jnp.arange(16)
    ones_src = jnp.broadcast_to((col < 8).astype(jnp.float32), (CE, 16))
    ones_dst = jnp.broadcast_to((col >= 8).astype(jnp.float32), (CE, 16))
    z16 = jnp.zeros((ZR, 16), jnp.float32)
    z64 = jnp.zeros((ZR, DH), jnp.float32)
    W3p = jnp.zeros((D_HID, 16), jnp.float32).at[:, :2].set(W3)
    b3p = jnp.zeros((1, 16), jnp.float32).at[0, :2].set(b3)

    deg_p = _deg_kernel(srcp, dstp, ones_src, ones_dst, z16)
    xw = _mm1(xpad, W1)
    ns, nd, g1 = _scale1(xw, deg_p)
    p1 = _prop128(g1, srcp, dstp, z64)
    g2 = _stage2(p1, nd, ns, b1.reshape(1, -1), W2)
    p2 = _prop128(g2, srcp, dstp, z64)
    g3 = _stage3(p2, nd, ns, b2.reshape(1, -1), W3p)
    p3 = _prop16(g3, srcp, dstp, z16)
    outp = _stage4(p3[0], p3[1], nd, b3p)
    return outp[:N, :2]


# speedup vs baseline: 7.1149x; 1.3623x over previous
import functools

import jax
import jax.numpy as jnp
from jax import lax
from jax.experimental import pallas as pl
from jax.experimental.pallas import tpu as pltpu
from jax.experimental.pallas import tpu_sc as plsc

N = 10000
E = 160000
D_IN = 256
D_HID = 128
DH = 64

NC = 2
NS = 16
NW = NC * NS
N_PAD = 10240
CE = 128
E_PAD = 163840
CHUNKS_W = E_PAD // (NW * CE)
CHUNKS_T = E_PAD // (NS * CE)
ROWS_T = N_PAD // NS
ZR = 64

NB = 2
NB128 = 4
NBATCH_W = CHUNKS_W // NB
NBATCH_T = CHUNKS_T // NB128
NPAIR_T = NBATCH_T // 2

BR = 512
GRID = N_PAD // BR


def _mesh():
    return plsc.VectorSubcoreMesh(core_axis_name="c", subcore_axis_name="s")


@functools.partial(
    pl.kernel,
    out_type=jax.ShapeDtypeStruct((NC, N_PAD, 16), jnp.float32),
    mesh=_mesh(),
    scratch_types=[
        pltpu.VMEM((CHUNKS_W, CE), jnp.int32),
        pltpu.VMEM((CHUNKS_W, CE), jnp.int32),
        pltpu.VMEM((CE, 16), jnp.float32),
        pltpu.VMEM((CE, 16), jnp.float32),
        pltpu.VMEM((ZR, 16), jnp.float32),
        pltpu.VMEM_SHARED((N_PAD, 16), jnp.float32),
    ],
    name="sc_degrees",
    compiler_params=pltpu.CompilerParams(use_tc_tiling_on_sc=False),
)
def _deg_kernel(src_hbm, dst_hbm, ones_src_hbm, ones_dst_hbm, zeros_hbm, deg_hbm,
                srcv, dstv, ones_s, ones_d, zeros_v, acc):
    cid = lax.axis_index("c")
    sid = lax.axis_index("s")
    wid = cid * NS + sid
    pltpu.sync_copy(ones_src_hbm, ones_s)
    pltpu.sync_copy(ones_dst_hbm, ones_d)
    pltpu.sync_copy(zeros_hbm, zeros_v)

    def zbody(i, _):
        pltpu.sync_copy(zeros_v, acc.at[pl.ds(sid * ROWS_T + i * ZR, ZR)])
        return 0
    lax.fori_loop(0, ROWS_T // ZR, zbody, 0)

    pltpu.sync_copy(src_hbm.at[pl.ds(wid * CHUNKS_W, CHUNKS_W)], srcv)
    pltpu.sync_copy(dst_hbm.at[pl.ds(wid * CHUNKS_W, CHUNKS_W)], dstv)
    plsc.subcore_barrier()

    def body(k, _):
        pltpu.sync_copy(ones_s, acc.at[srcv.at[k]], add=True)
        pltpu.sync_copy(ones_d, acc.at[dstv.at[k]], add=True)
        return 0
    lax.fori_loop(0, CHUNKS_W, body, 0)
    plsc.subcore_barrier()

    r0 = sid * ROWS_T
    pltpu.sync_copy(acc.at[pl.ds(r0, ROWS_T)], deg_hbm.at[cid, pl.ds(r0, ROWS_T)])


@functools.partial(
    pl.kernel,
    out_type=jax.ShapeDtypeStruct((NC, N_PAD, DH), jnp.bfloat16),
    mesh=_mesh(),
    scratch_types=[
        pltpu.VMEM((CHUNKS_T, CE), jnp.int32),
        pltpu.VMEM((CHUNKS_T, CE), jnp.int32),
        pltpu.VMEM((NB128, CE, DH), jnp.bfloat16),
        pltpu.VMEM((NB128, CE, DH), jnp.bfloat16),
        pltpu.VMEM((ZR, DH), jnp.bfloat16),
        pltpu.SemaphoreType.DMA,
        pltpu.SemaphoreType.DMA,
        pltpu.SemaphoreType.DMA,
        pltpu.SemaphoreType.DMA,
        pltpu.VMEM_SHARED((N_PAD, DH), jnp.bfloat16),
    ],
    name="sc_prop_128",
    compiler_params=pltpu.CompilerParams(use_tc_tiling_on_sc=False),
)
def _prop128(tbl_hbm, src_hbm, dst_hbm, zeros_hbm, out_hbm,
             srcv, dstv, rows_a, rows_b, zeros_v,
             gsem_a, gsem_b, ssem_a, ssem_b, acc):
    cid = lax.axis_index("c")
    sid = lax.axis_index("s")
    tbl = tbl_hbm.at[cid]
    pltpu.sync_copy(zeros_hbm, zeros_v)

    def zbody(i, _):
        pltpu.sync_copy(zeros_v, acc.at[pl.ds(sid * ROWS_T + i * ZR, ZR)])
        return 0
    lax.fori_loop(0, ROWS_T // ZR, zbody, 0)

    pltpu.sync_copy(src_hbm.at[pl.ds(sid * CHUNKS_T, CHUNKS_T)], srcv)
    pltpu.sync_copy(dst_hbm.at[pl.ds(sid * CHUNKS_T, CHUNKS_T)], dstv)
    plsc.subcore_barrier()

    def fire_g(rows, sem, t):
        for b in range(NB128):
            pltpu.async_copy(tbl.at[srcv.at[t * NB128 + b]], rows.at[b], sem)

    def drain_g(rows, sem, t):
        for b in range(NB128):
            pltpu.make_async_copy(
                tbl.at[srcv.at[t * NB128 + b]], rows.at[b], sem).wait()

    def fire_s(rows, sem, t):
        for b in range(NB128):
            pltpu.async_copy(rows.at[b], acc.at[dstv.at[t * NB128 + b]], sem,
                             add=True)

    def drain_s(rows, sem, t):
        for b in range(NB128):
            pltpu.make_async_copy(
                rows.at[b], acc.at[dstv.at[t * NB128 + b]], sem).wait()

    fire_g(rows_a, gsem_a, 0)

    def body(h, _):
        ta = 2 * h
        tb = 2 * h + 1
        drain_g(rows_a, gsem_a, ta)
        fire_s(rows_a, ssem_a, ta)

        @pl.when(h > 0)
        def _():
            drain_s(rows_b, ssem_b, tb - 2)
        fire_g(rows_b, gsem_b, tb)
        drain_g(rows_b, gsem_b, tb)
        fire_s(rows_b, ssem_b, tb)

        drain_s(rows_a, ssem_a, ta)

        @pl.when(h < NPAIR_T - 1)
        def _():
            fire_g(rows_a, gsem_a, ta + 2)
        return 0
    lax.fori_loop(0, NPAIR_T, body, 0)
    drain_s(rows_b, ssem_b, 2 * NPAIR_T - 1)
    plsc.subcore_barrier()

    r0 = sid * ROWS_T
    pltpu.sync_copy(acc.at[pl.ds(r0, ROWS_T)],
                    out_hbm.at[cid, pl.ds(r0, ROWS_T)])


@functools.partial(
    pl.kernel,
    out_type=jax.ShapeDtypeStruct((NC, N_PAD, 16), jnp.float32),
    mesh=_mesh(),
    scratch_types=[
        pltpu.VMEM((CHUNKS_W, CE), jnp.int32),
        pltpu.VMEM((CHUNKS_W, CE), jnp.int32),
        pltpu.VMEM((NB, CE, 16), jnp.float32),
        pltpu.VMEM((ZR, 16), jnp.float32),
        pltpu.SemaphoreType.DMA,
        pltpu.VMEM_SHARED((N_PAD, 16), jnp.float32),
    ],
    name="sc_prop_16",
    compiler_params=pltpu.CompilerParams(use_tc_tiling_on_sc=False),
)
def _prop16(tbl_hbm, src_hbm, dst_hbm, zeros_hbm, out_hbm,
            srcv, dstv, rows_v, zeros_v, gsem, acc):
    cid = lax.axis_index("c")
    sid = lax.axis_index("s")
    wid = cid * NS + sid
    pltpu.sync_copy(zeros_hbm, zeros_v)

    def zbody(i, _):
        pltpu.sync_copy(zeros_v, acc.at[pl.ds(sid * ROWS_T + i * ZR, ZR)])
        return 0
    lax.fori_loop(0, ROWS_T // ZR, zbody, 0)

    pltpu.sync_copy(src_hbm.at[pl.ds(wid * CHUNKS_W, CHUNKS_W)], srcv)
    pltpu.sync_copy(dst_hbm.at[pl.ds(wid * CHUNKS_W, CHUNKS_W)], dstv)
    plsc.subcore_barrier()

    def body(g, _):
        for b in range(NB):
            pltpu.async_copy(tbl_hbm.at[srcv.at[g * NB + b]],
                             rows_v.at[b], gsem)
        for b in range(NB):
            pltpu.make_async_copy(tbl_hbm.at[srcv.at[g * NB + b]],
                                  rows_v.at[b], gsem).wait()
            pltpu.sync_copy(rows_v.at[b], acc.at[dstv.at[g * NB + b]],
                            add=True)
        return 0
    lax.fori_loop(0, NBATCH_W, body, 0)
    plsc.subcore_barrier()

    r0 = sid * ROWS_T
    pltpu.sync_copy(acc.at[pl.ds(r0, ROWS_T)], out_hbm.at[cid, pl.ds(r0, ROWS_T)])


def _split_halves(o_ref, res):
    o_ref[0, :, :] = res[:, :DH]
    o_ref[1, :, :] = res[:, DH:]


def _mm1_body(x_ref, w_ref, o_ref):
    o_ref[...] = jnp.dot(x_ref[...], w_ref[...],
                         preferred_element_type=jnp.float32)


def _mm1(x, w):
    return pl.pallas_call(
        _mm1_body,
        grid=(GRID,),
        in_specs=[
            pl.BlockSpec((BR, D_IN), lambda i: (i, 0)),
            pl.BlockSpec((D_IN, D_HID), lambda i: (0, 0)),
        ],
        out_specs=pl.BlockSpec((BR, D_HID), lambda i: (i, 0)),
        out_shape=jax.ShapeDtypeStruct((N_PAD, D_HID), jnp.float32),
    )(x, w)


def _scale1_body(xw_ref, d0, d1, ns_ref, nd_ref, o_ref):
    deg = d0[...] + d1[...]
    ns = lax.rsqrt(jnp.maximum(deg[:, 0:1], 1.0))
    nd = lax.rsqrt(jnp.maximum(deg[:, 8:9], 1.0))
    ns_ref[...] = ns * jnp.ones((1, 16), jnp.float32)
    nd_ref[...] = nd * jnp.ones((1, 16), jnp.float32)
    _split_halves(o_ref, (xw_ref[...] * ns).astype(jnp.bfloat16))


def _scale1(xw, deg_p):
    return pl.pallas_call(
        _scale1_body,
        grid=(GRID,),
        in_specs=[
            pl.BlockSpec((BR, D_HID), lambda i: (i, 0)),
            pl.BlockSpec((BR, 16), lambda i: (i, 0)),
            pl.BlockSpec((BR, 16), lambda i: (i, 0)),
        ],
        out_specs=(pl.BlockSpec((BR, 16), lambda i: (i, 0)),
                   pl.BlockSpec((BR, 16), lambda i: (i, 0)),
                   pl.BlockSpec((NC, BR, DH), lambda i: (0, i, 0))),
        out_shape=(jax.ShapeDtypeStruct((N_PAD, 16), jnp.float32),
                   jax.ShapeDtypeStruct((N_PAD, 16), jnp.float32),
                   jax.ShapeDtypeStruct((NC, N_PAD, DH), jnp.bfloat16)),
    )(xw, deg_p[0], deg_p[1])


def _stage2_body(p_ref, nd_ref, ns_ref, b_ref, w_ref, o_ref):
    p = jnp.concatenate([p_ref[0], p_ref[1]], axis=1).astype(jnp.float32)
    h = p * nd_ref[:, 0:1] + b_ref[...]
    res = jnp.dot(h, w_ref[...],
                  preferred_element_type=jnp.float32) * ns_ref[:, 0:1]
    _split_halves(o_ref, res.astype(jnp.bfloat16))


def _stage2(p, nd, ns, b, w):
    return pl.pallas_call(
        _stage2_body,
        grid=(GRID,),
        in_specs=[
            pl.BlockSpec((NC, BR, DH), lambda i: (0, i, 0)),
            pl.BlockSpec((BR, 16), lambda i: (i, 0)),
            pl.BlockSpec((BR, 16), lambda i: (i, 0)),
            pl.BlockSpec((1, D_HID), lambda i: (0, 0)),
            pl.BlockSpec((D_HID, D_HID), lambda i: (0, 0)),
        ],
        out_specs=pl.BlockSpec((NC, BR, DH), lambda i: (0, i, 0)),
        out_shape=jax.ShapeDtypeStruct((NC, N_PAD, DH), jnp.bfloat16),
    )(p, nd, ns, b, w)


def _stage3_body(p_ref, nd_ref, ns_ref, b_ref, w_ref, o_ref):
    p = jnp.concatenate([p_ref[0], p_ref[1]], axis=1).astype(jnp.float32)
    h = p * nd_ref[:, 0:1] + b_ref[...]
    o_ref[...] = jnp.dot(h, w_ref[...],
                         preferred_element_type=jnp.float32) * ns_ref[:, 0:1]


def _stage3(p, nd, ns, b, w):
    return pl.pallas_call(
        _stage3_body,
        grid=(GRID,),
        in_specs=[
            pl.BlockSpec((NC, BR, DH), lambda i: (0, i, 0)),
            pl.BlockSpec((BR, 16), lambda i: (i, 0)),
            pl.BlockSpec((BR, 16), lambda i: (i, 0)),
            pl.BlockSpec((1, D_HID), lambda i: (0, 0)),
            pl.BlockSpec((D_HID, 16), lambda i: (0, 0)),
        ],
        out_specs=pl.BlockSpec((BR, 16), lambda i: (i, 0)),
        out_shape=jax.ShapeDtypeStruct((N_PAD, 16), jnp.float32),
    )(p, nd, ns, b, w)


def _stage4_body(p0_ref, p1_ref, nd_ref, b_ref, o_ref):
    o_ref[...] = (p0_ref[...] + p1_ref[...]) * nd_ref[:, 0:1] + b_ref[...]


def _stage4(p0, p1, nd, b):
    return pl.pallas_call(
        _stage4_body,
        grid=(GRID,),
        in_specs=[
            pl.BlockSpec((BR, 16), lambda i: (i, 0)),
            pl.BlockSpec((BR, 16), lambda i: (i, 0)),
            pl.BlockSpec((BR, 16), lambda i: (i, 0)),
            pl.BlockSpec((1, 16), lambda i: (0, 0)),
        ],
        out_specs=pl.BlockSpec((BR, 16), lambda i: (i, 0)),
        out_shape=jax.ShapeDtypeStruct((N_PAD, 16), jnp.float32),
    )(p0, p1, nd, b)


def kernel(feature, edge_index, W1, b1, W2, b2, W3, b3):
    src = edge_index[0]
    dst = edge_index[1]
    fill = jnp.full((E_PAD - E,), N, jnp.int32)
    srcp = jnp.concatenate([src, fill]).reshape(E_PAD // CE, CE)
    dstp = jnp.concatenate([dst, fill]).reshape(E_PAD // CE, CE)
    xpad = jnp.zeros((N_PAD, D_IN), jnp.float32).at[:N].set(feature)
    col = jnp.arange(16)
    ones_src = jnp.broadcast_to((col < 8).astype(jnp.float32), (CE, 16))
    ones_dst = jnp.broadcast_to((col >= 8).astype(jnp.float32), (CE, 16))
    z16 = jnp.zeros((ZR, 16), jnp.float32)
    z64 = jnp.zeros((ZR, DH), jnp.bfloat16)
    W3p = jnp.zeros((D_HID, 16), jnp.float32).at[:, :2].set(W3)
    b3p = jnp.zeros((1, 16), jnp.float32).at[0, :2].set(b3)

    deg_p = _deg_kernel(srcp, dstp, ones_src, ones_dst, z16)
    xw = _mm1(xpad, W1)
    ns, nd, g1 = _scale1(xw, deg_p)
    p1 = _prop128(g1, srcp, dstp, z64)
    g2 = _stage2(p1, nd, ns, b1.reshape(1, -1), W2)
    p2 = _prop128(g2, srcp, dstp, z64)
    g3 = _stage3(p2, nd, ns, b2.reshape(1, -1), W3p)
    p3 = _prop16(g3, srcp, dstp, z16)
    outp = _stage4(p3[0], p3[1], nd, b3p)
    return outp[:N, :2]
